# Initial kernel scaffold; baseline (speedup 1.0000x reference)
import functools

import jax
import jax.numpy as jnp
from jax import lax
from jax.experimental import pallas as pl
from jax.experimental.pallas import tpu as pltpu
from jax.experimental.pallas import tpu_sc as plsc

N = 10000
E = 320000
D = 128
NC = 2
NS = 16
NW = NC * NS
EPT = E // NW
K = 80
CHUNKS = EPT // K
RPS = N // NS
ZC = 125
HW = 16

_vmesh = functools.partial(
    plsc.VectorSubcoreMesh, core_axis_name="c", subcore_axis_name="s")


def _sc_hist(dst, ones, zeros16):
    @functools.partial(
        pl.kernel,
        out_type=jax.ShapeDtypeStruct((NC, N, HW), jnp.float32),
        mesh=_vmesh(),
        scratch_types=[
            pltpu.VMEM((K,), jnp.int32),
            pltpu.VMEM((K, HW), jnp.float32),
            pltpu.VMEM((ZC, HW), jnp.float32),
            pltpu.VMEM_SHARED((N, HW), jnp.float32),
        ],
    )
    def k(dst_hbm, ones_hbm, zeros_hbm, out_hbm, idx_v, ones_v, zbuf, acc):
        cid = lax.axis_index("c")
        sid = lax.axis_index("s")
        wid = sid * NC + cid

        pltpu.sync_copy(ones_hbm, ones_v)
        pltpu.sync_copy(zeros_hbm, zbuf)

        @pl.loop(0, RPS // ZC)
        def _(j):
            pltpu.sync_copy(zbuf, acc.at[pl.ds(sid * RPS + j * ZC, ZC)])

        plsc.subcore_barrier()

        @pl.loop(0, CHUNKS)
        def _(it):
            base = wid * EPT + it * K
            pltpu.sync_copy(dst_hbm.at[pl.ds(base, K)], idx_v)
            pltpu.sync_copy(ones_v, acc.at[idx_v], add=True)

        plsc.subcore_barrier()

        @pl.loop(0, RPS // ZC)
        def _(j):
            r0 = sid * RPS + j * ZC
            pltpu.sync_copy(acc.at[pl.ds(r0, ZC)], zbuf)
            pltpu.sync_copy(zbuf, out_hbm.at[cid, pl.ds(r0, ZC)])

    return k(dst, ones, zeros16)


def _sc_aggregate(src, dst, table, zeros128):
    @functools.partial(
        pl.kernel,
        out_type=jax.ShapeDtypeStruct((NC, N, D), jnp.float32),
        mesh=_vmesh(),
        scratch_types=[
            pltpu.VMEM((K,), jnp.int32),
            pltpu.VMEM((K,), jnp.int32),
            pltpu.VMEM((K, D), jnp.float32),
            pltpu.VMEM((ZC, D), jnp.float32),
            pltpu.VMEM_SHARED((N, D), jnp.float32),
        ],
    )
    def k(src_hbm, dst_hbm, table_hbm, zeros_hbm, out_hbm,
          idx_s, idx_d, rows, zbuf, acc):
        cid = lax.axis_index("c")
        sid = lax.axis_index("s")
        wid = sid * NC + cid

        pltpu.sync_copy(zeros_hbm, zbuf)

        @pl.loop(0, RPS // ZC)
        def _(j):
            pltpu.sync_copy(zbuf, acc.at[pl.ds(sid * RPS + j * ZC, ZC)])

        plsc.subcore_barrier()

        @pl.loop(0, CHUNKS)
        def _(it):
            base = wid * EPT + it * K
            pltpu.sync_copy(src_hbm.at[pl.ds(base, K)], idx_s)
            pltpu.sync_copy(dst_hbm.at[pl.ds(base, K)], idx_d)
            pltpu.sync_copy(table_hbm.at[idx_s], rows)
            pltpu.sync_copy(rows, acc.at[idx_d], add=True)

        plsc.subcore_barrier()

        @pl.loop(0, RPS // ZC)
        def _(j):
            r0 = sid * RPS + j * ZC
            pltpu.sync_copy(acc.at[pl.ds(r0, ZC)], zbuf)
            pltpu.sync_copy(zbuf, out_hbm.at[cid, pl.ds(r0, ZC)])

    return k(src, dst, table, zeros128)


_TCB = 1000


def _dinv_of(degp_ref):
    deg = degp_ref[0, :, 0:1] + degp_ref[1, :, 0:1] + 1.0
    return lax.rsqrt(deg)


def _xs1_body(x_ref, degp_ref, w_ref, o_ref):
    dinv = _dinv_of(degp_ref)
    xw = jnp.dot(x_ref[...], w_ref[...], preferred_element_type=jnp.float32)
    o_ref[...] = xw * dinv


def _mid_body(p1_ref, degp_ref, xs1_ref, b1_ref, o_ref):
    dinv = _dinv_of(degp_ref)
    xs1 = xs1_ref[...]
    h = jnp.maximum(dinv * (p1_ref[0] + p1_ref[1] + xs1) + b1_ref[...], 0.0)
    o_ref[...] = dinv * h


def _fin_body(p2_ref, degp_ref, xs2_ref, wml_ref, bml_ref, o_ref):
    dinv = _dinv_of(degp_ref)
    agg = dinv * (p2_ref[0] + p2_ref[1] + xs2_ref[...])
    o_ref[...] = (
        jnp.dot(agg, wml_ref[...], preferred_element_type=jnp.float32)
        + bml_ref[...])


_ROWS = pl.BlockSpec((_TCB, D), lambda i: (i, 0))
_PARTS = pl.BlockSpec((NC, _TCB, D), lambda i: (0, i, 0))
_DEGP = pl.BlockSpec((NC, _TCB, HW), lambda i: (0, i, 0))
_WMAT = pl.BlockSpec((D, D), lambda i: (0, 0))
_BVEC = pl.BlockSpec((1, D), lambda i: (0, 0))


def _tc_call(body, in_specs):
    return pl.pallas_call(
        body, grid=(N // _TCB,), in_specs=in_specs, out_specs=_ROWS,
        out_shape=jax.ShapeDtypeStruct((N, D), jnp.float32))


def kernel(x, edge_index, W1, b1, W_mu, b_mu, W_lv, b_lv):
    src = edge_index[0]
    dst = edge_index[1]

    ones = jnp.ones((K, HW), jnp.float32)
    zeros16 = jnp.zeros((ZC, HW), jnp.float32)
    zeros128 = jnp.zeros((ZC, D), jnp.float32)
    wml = jnp.concatenate([W_mu, W_lv], axis=1)
    bml = jnp.concatenate([b_mu, b_lv])[None, :]
    b1_2d = b1[None, :]

    degp = _sc_hist(dst, ones, zeros16)
    xs1 = _tc_call(_xs1_body, [_ROWS, _DEGP, _WMAT])(x, degp, W1)
    p1 = _sc_aggregate(src, dst, xs1, zeros128)
    xs2 = _tc_call(_mid_body, [_PARTS, _DEGP, _ROWS, _BVEC])(
        p1, degp, xs1, b1_2d)
    p2 = _sc_aggregate(src, dst, xs2, zeros128)
    out = _tc_call(_fin_body, [_PARTS, _DEGP, _ROWS, _WMAT, _BVEC])(
        p2, degp, xs2, wml, bml)
    return out[:, :64], out[:, 64:]


# speedup vs baseline: 15.3556x; 15.3556x over previous
import functools

import jax
import jax.numpy as jnp
from jax import lax
from jax.experimental import pallas as pl
from jax.experimental.pallas import tpu as pltpu
from jax.experimental.pallas import tpu_sc as plsc

N = 10000
NP = 10240
E = 320000
D = 128
NC = 2
NS = 16
NW = NC * NS
EPT = E // NW
K = 80
CHUNKS = EPT // K
RPS = NP // NS
ZC = 128
HW = 16

_vmesh = functools.partial(
    plsc.VectorSubcoreMesh, core_axis_name="c", subcore_axis_name="s")


def _sc_hist(dst, ones, zeros16):
    @functools.partial(
        pl.kernel,
        out_type=jax.ShapeDtypeStruct((NC, NP, HW), jnp.float32),
        mesh=_vmesh(),
        scratch_types=[
            pltpu.VMEM((K,), jnp.int32),
            pltpu.VMEM((K, HW), jnp.float32),
            pltpu.VMEM((ZC, HW), jnp.float32),
            pltpu.VMEM_SHARED((NP, HW), jnp.float32),
        ],
    )
    def k(dst_hbm, ones_hbm, zeros_hbm, out_hbm, idx_v, ones_v, zbuf, acc):
        cid = lax.axis_index("c")
        sid = lax.axis_index("s")
        wid = sid * NC + cid

        pltpu.sync_copy(ones_hbm, ones_v)
        pltpu.sync_copy(zeros_hbm, zbuf)

        @pl.loop(0, RPS // ZC)
        def _(j):
            pltpu.sync_copy(zbuf, acc.at[pl.ds(sid * RPS + j * ZC, ZC)])

        plsc.subcore_barrier()

        @pl.loop(0, CHUNKS)
        def _(it):
            base = wid * EPT + it * K
            pltpu.sync_copy(dst_hbm.at[pl.ds(base, K)], idx_v)
            pltpu.sync_copy(ones_v, acc.at[idx_v], add=True)

        plsc.subcore_barrier()

        @pl.loop(0, RPS // ZC)
        def _(j):
            r0 = sid * RPS + j * ZC
            pltpu.sync_copy(acc.at[pl.ds(r0, ZC)], zbuf)
            pltpu.sync_copy(zbuf, out_hbm.at[cid, pl.ds(r0, ZC)])

    return k(dst, ones, zeros16)


def _sc_aggregate(src, dst, table, zeros128):
    @functools.partial(
        pl.kernel,
        out_type=jax.ShapeDtypeStruct((NC, NP, D), jnp.float32),
        mesh=_vmesh(),
        scratch_types=[
            pltpu.VMEM((K,), jnp.int32),
            pltpu.VMEM((K,), jnp.int32),
            pltpu.VMEM((K, D), jnp.float32),
            pltpu.VMEM((ZC, D), jnp.float32),
            pltpu.VMEM_SHARED((NP, D), jnp.float32),
        ],
    )
    def k(src_hbm, dst_hbm, table_hbm, zeros_hbm, out_hbm,
          idx_s, idx_d, rows, zbuf, acc):
        cid = lax.axis_index("c")
        sid = lax.axis_index("s")
        wid = sid * NC + cid

        pltpu.sync_copy(zeros_hbm, zbuf)

        @pl.loop(0, RPS // ZC)
        def _(j):
            pltpu.sync_copy(zbuf, acc.at[pl.ds(sid * RPS + j * ZC, ZC)])

        plsc.subcore_barrier()

        @pl.loop(0, CHUNKS)
        def _(it):
            base = wid * EPT + it * K
            pltpu.sync_copy(src_hbm.at[pl.ds(base, K)], idx_s)
            pltpu.sync_copy(dst_hbm.at[pl.ds(base, K)], idx_d)
            pltpu.sync_copy(table_hbm.at[idx_s], rows)
            pltpu.sync_copy(rows, acc.at[idx_d], add=True)

        plsc.subcore_barrier()

        @pl.loop(0, RPS // ZC)
        def _(j):
            r0 = sid * RPS + j * ZC
            pltpu.sync_copy(acc.at[pl.ds(r0, ZC)], zbuf)
            pltpu.sync_copy(zbuf, out_hbm.at[cid, pl.ds(r0, ZC)])

    return k(src, dst, table, zeros128)


_TCB = 1000


def _dinv_of(degp_ref):
    deg = degp_ref[0, :, 0:1] + degp_ref[1, :, 0:1] + 1.0
    return lax.rsqrt(deg)


def _xs1_body(x_ref, degp_ref, w_ref, o_ref):
    dinv = _dinv_of(degp_ref)
    xw = jnp.dot(x_ref[...], w_ref[...], preferred_element_type=jnp.float32)
    o_ref[...] = xw * dinv


def _mid_body(p1_ref, degp_ref, xs1_ref, b1_ref, o_ref):
    dinv = _dinv_of(degp_ref)
    xs1 = xs1_ref[...]
    h = jnp.maximum(dinv * (p1_ref[0] + p1_ref[1] + xs1) + b1_ref[...], 0.0)
    o_ref[...] = dinv * h


def _fin_body(p2_ref, degp_ref, xs2_ref, wml_ref, bml_ref, o_ref):
    dinv = _dinv_of(degp_ref)
    agg = dinv * (p2_ref[0] + p2_ref[1] + xs2_ref[...])
    o_ref[...] = (
        jnp.dot(agg, wml_ref[...], preferred_element_type=jnp.float32)
        + bml_ref[...])


_ROWS = pl.BlockSpec((_TCB, D), lambda i: (i, 0))
_PARTS = pl.BlockSpec((NC, _TCB, D), lambda i: (0, i, 0))
_DEGP = pl.BlockSpec((NC, _TCB, HW), lambda i: (0, i, 0))
_WMAT = pl.BlockSpec((D, D), lambda i: (0, 0))
_BVEC = pl.BlockSpec((1, D), lambda i: (0, 0))


def _tc_call(body, in_specs):
    return pl.pallas_call(
        body, grid=(N // _TCB,), in_specs=in_specs, out_specs=_ROWS,
        out_shape=jax.ShapeDtypeStruct((N, D), jnp.float32))


def kernel(x, edge_index, W1, b1, W_mu, b_mu, W_lv, b_lv):
    src = edge_index[0]
    dst = edge_index[1]

    ones = jnp.ones((K, HW), jnp.float32)
    zeros16 = jnp.zeros((ZC, HW), jnp.float32)
    zeros128 = jnp.zeros((ZC, D), jnp.float32)
    wml = jnp.concatenate([W_mu, W_lv], axis=1)
    bml = jnp.concatenate([b_mu, b_lv])[None, :]
    b1_2d = b1[None, :]

    degp = _sc_hist(dst, ones, zeros16)
    xs1 = _tc_call(_xs1_body, [_ROWS, _DEGP, _WMAT])(x, degp, W1)
    p1 = _sc_aggregate(src, dst, xs1, zeros128)
    xs2 = _tc_call(_mid_body, [_PARTS, _DEGP, _ROWS, _BVEC])(
        p1, degp, xs1, b1_2d)
    p2 = _sc_aggregate(src, dst, xs2, zeros128)
    out = _tc_call(_fin_body, [_PARTS, _DEGP, _ROWS, _WMAT, _BVEC])(
        p2, degp, xs2, wml, bml)
    return out[:, :64], out[:, 64:]


# speedup vs baseline: 22.5621x; 1.4693x over previous
import functools

import jax
import jax.numpy as jnp
from jax import lax
from jax.experimental import pallas as pl
from jax.experimental.pallas import tpu as pltpu
from jax.experimental.pallas import tpu_sc as plsc

N = 10000
NP = 10240
E = 320000
D = 128
NC = 2
NS = 16
NW = NC * NS
EPT = E // NW
K = 80
CHUNKS = EPT // K
NB = 4
HNB = 5
RPS = NP // NS
ZC = 128
HW = 16

_vmesh = functools.partial(
    plsc.VectorSubcoreMesh, core_axis_name="c", subcore_axis_name="s")


def _sc_hist(dst, ones, zeros16):
    @functools.partial(
        pl.kernel,
        out_type=jax.ShapeDtypeStruct((NC, NP, HW), jnp.float32),
        mesh=_vmesh(),
        scratch_types=(
            [pltpu.VMEM((K,), jnp.int32)] * HNB
            + [pltpu.VMEM((K, HW), jnp.float32),
               pltpu.VMEM((ZC, HW), jnp.float32),
               pltpu.VMEM_SHARED((NP, HW), jnp.float32)]
            + [pltpu.SemaphoreType.DMA] * HNB
        ),
    )
    def k(dst_hbm, ones_hbm, zeros_hbm, out_hbm, *scr):
        idx_v = scr[:HNB]
        ones_v, zbuf, acc = scr[HNB:HNB + 3]
        sem_d = scr[HNB + 3:HNB + 3 + HNB]
        cid = lax.axis_index("c")
        sid = lax.axis_index("s")
        wid = sid * NC + cid

        pltpu.sync_copy(ones_hbm, ones_v)
        pltpu.sync_copy(zeros_hbm, zbuf)

        @pl.loop(0, RPS // ZC)
        def _(j):
            pltpu.sync_copy(zbuf, acc.at[pl.ds(sid * RPS + j * ZC, ZC)])

        plsc.subcore_barrier()

        @pl.loop(0, CHUNKS)
        def _(it):
            base = wid * EPT + it * K
            pltpu.sync_copy(dst_hbm.at[pl.ds(base, K)], idx_v[0])
            pltpu.sync_copy(ones_v, acc.at[idx_v[0]], add=True)

        plsc.subcore_barrier()

        @pl.loop(0, RPS // ZC)
        def _(j):
            r0 = sid * RPS + j * ZC
            pltpu.sync_copy(acc.at[pl.ds(r0, ZC)], zbuf)
            pltpu.sync_copy(zbuf, out_hbm.at[cid, pl.ds(r0, ZC)])

    return k(dst, ones, zeros16)


def _sc_aggregate(src, dst, table, zeros128):
    @functools.partial(
        pl.kernel,
        out_type=jax.ShapeDtypeStruct((NC, NP, D), jnp.float32),
        mesh=_vmesh(),
        scratch_types=(
            [pltpu.VMEM((K,), jnp.int32)] * (2 * NB)
            + [pltpu.VMEM((K, D), jnp.float32)] * NB
            + [pltpu.VMEM_SHARED((NP, D), jnp.float32)]
            + [pltpu.SemaphoreType.DMA] * (3 * NB)
        ),
    )
    def k(src_hbm, dst_hbm, table_hbm, zeros_hbm, out_hbm, *scr):
        idx_s = scr[:NB]
        idx_d = scr[NB:2 * NB]
        rows = scr[2 * NB:3 * NB]
        acc = scr[3 * NB]
        sem_s = scr[3 * NB + 1:4 * NB + 1]
        sem_d = scr[4 * NB + 1:5 * NB + 1]
        sem_g = scr[5 * NB + 1:6 * NB + 1]
        cid = lax.axis_index("c")
        sid = lax.axis_index("s")
        wid = sid * NC + cid

        pltpu.sync_copy(zeros_hbm, rows[0])

        @pl.loop(0, RPS // K)
        def _(j):
            pltpu.sync_copy(rows[0], acc.at[pl.ds(sid * RPS + j * K, K)])

        plsc.subcore_barrier()

        def _load_idx(i, b):
            base = wid * EPT + i * K
            pltpu.sync_copy(src_hbm.at[pl.ds(base, K)], idx_s[b])
            pltpu.sync_copy(dst_hbm.at[pl.ds(base, K)], idx_d[b])

        _load_idx(0, 0)
        pltpu.async_copy(table_hbm.at[idx_s[0]], rows[0], sem_g[0])

        @pl.loop(0, (CHUNKS - 1) // 2)
        def _(o):
            for b in range(2):
                i = o * 2 + b
                nb = 1 - b
                _load_idx(i + 1, nb)
                pltpu.make_async_copy(
                    table_hbm.at[idx_s[b]], rows[b], sem_g[b]).wait()
                pltpu.async_copy(
                    table_hbm.at[idx_s[nb]], rows[nb], sem_g[nb])
                pltpu.sync_copy(rows[b], acc.at[idx_d[b]], add=True)

        lb = (CHUNKS - 1) % 2
        pltpu.make_async_copy(
            table_hbm.at[idx_s[lb]], rows[lb], sem_g[lb]).wait()
        pltpu.sync_copy(rows[lb], acc.at[idx_d[lb]], add=True)

        plsc.subcore_barrier()

        @pl.loop(0, RPS // K)
        def _(j):
            r0 = sid * RPS + j * K
            pltpu.sync_copy(acc.at[pl.ds(r0, K)], rows[0])
            pltpu.sync_copy(rows[0], out_hbm.at[cid, pl.ds(r0, K)])

    return k(src, dst, table, zeros128)


_TCB = 1000


def _dinv_of(degp_ref):
    deg = degp_ref[0, :, 0:1] + degp_ref[1, :, 0:1] + 1.0
    return lax.rsqrt(deg)


def _xs1_body(x_ref, degp_ref, w_ref, o_ref):
    dinv = _dinv_of(degp_ref)
    xw = jnp.dot(x_ref[...], w_ref[...], preferred_element_type=jnp.float32)
    o_ref[...] = xw * dinv


def _mid_body(p1_ref, degp_ref, xs1_ref, b1_ref, o_ref):
    dinv = _dinv_of(degp_ref)
    xs1 = xs1_ref[...]
    h = jnp.maximum(dinv * (p1_ref[0] + p1_ref[1] + xs1) + b1_ref[...], 0.0)
    o_ref[...] = dinv * h


def _fin_body(p2_ref, degp_ref, xs2_ref, wml_ref, bml_ref, o_ref):
    dinv = _dinv_of(degp_ref)
    agg = dinv * (p2_ref[0] + p2_ref[1] + xs2_ref[...])
    o_ref[...] = (
        jnp.dot(agg, wml_ref[...], preferred_element_type=jnp.float32)
        + bml_ref[...])


_ROWS = pl.BlockSpec((_TCB, D), lambda i: (i, 0))
_PARTS = pl.BlockSpec((NC, _TCB, D), lambda i: (0, i, 0))
_DEGP = pl.BlockSpec((NC, _TCB, HW), lambda i: (0, i, 0))
_WMAT = pl.BlockSpec((D, D), lambda i: (0, 0))
_BVEC = pl.BlockSpec((1, D), lambda i: (0, 0))


def _tc_call(body, in_specs):
    return pl.pallas_call(
        body, grid=(N // _TCB,), in_specs=in_specs, out_specs=_ROWS,
        out_shape=jax.ShapeDtypeStruct((N, D), jnp.float32))


def kernel(x, edge_index, W1, b1, W_mu, b_mu, W_lv, b_lv):
    src = edge_index[0]
    dst = edge_index[1]

    ones = jnp.ones((K, HW), jnp.float32)
    zeros16 = jnp.zeros((ZC, HW), jnp.float32)
    zeros128 = jnp.zeros((K, D), jnp.float32)
    wml = jnp.concatenate([W_mu, W_lv], axis=1)
    bml = jnp.concatenate([b_mu, b_lv])[None, :]
    b1_2d = b1[None, :]

    degp = _sc_hist(dst, ones, zeros16)
    xs1 = _tc_call(_xs1_body, [_ROWS, _DEGP, _WMAT])(x, degp, W1)
    p1 = _sc_aggregate(src, dst, xs1, zeros128)
    xs2 = _tc_call(_mid_body, [_PARTS, _DEGP, _ROWS, _BVEC])(
        p1, degp, xs1, b1_2d)
    p2 = _sc_aggregate(src, dst, xs2, zeros128)
    out = _tc_call(_fin_body, [_PARTS, _DEGP, _ROWS, _WMAT, _BVEC])(
        p2, degp, xs2, wml, bml)
    return out[:, :64], out[:, 64:]
